# baseline (speedup 1.0000x reference)
import functools

import jax
import jax.numpy as jnp
from jax import lax
from jax.experimental import pallas as pl
from jax.experimental.pallas import tpu as pltpu
from jax.experimental.pallas import tpu_sc as plsc

N = 10000
E = 160000
DIN = 256
H = 512
G = 64
NCLS = 5

NPAD = 10240
BN_ = 512
NBLK = NPAD // BN_
NC, NS = 2, 16
K = 128
CH = 40
CH_A, CH_B = 78, 2
EPAD = NC * NS * K * CH
STRIPE = NPAD // NS
GARBAGE = 10016

_F32 = jnp.float32


def _sc_scatter_rows(S, CH0=CH, CH1=CH):
    mesh = plsc.VectorSubcoreMesh(core_axis_name="c", subcore_axis_name="s")
    out_type = [jax.ShapeDtypeStruct((NC, NPAD, 128), _F32) for _ in range(S)]
    scratch = [
        pltpu.VMEM_SHARED((NPAD, 128), _F32),
        pltpu.VMEM((2, K), jnp.int32),
        pltpu.VMEM((2, K), jnp.int32),
        pltpu.VMEM((2, K, 128), _F32),
        pltpu.SemaphoreType.DMA,
        pltpu.SemaphoreType.DMA,
        pltpu.SemaphoreType.DMA,
        pltpu.SemaphoreType.DMA,
    ]

    @functools.partial(pl.kernel, out_type=out_type, mesh=mesh,
                       scratch_types=scratch)
    def body(rowi, coli, zeros_hbm, *rest):
        tabs = rest[:S]
        outs = rest[S:2 * S]
        acc, rowv, colv, datav, si0, si1, sg0, sg1 = rest[2 * S:]
        semi = (si0, si1)
        semg = (sg0, sg1)
        c = lax.axis_index("c")
        s = lax.axis_index("s")
        ebase = jnp.where(c == 0, s * (K * CH0),
                          NS * (K * CH0) + s * (K * CH1))
        cht = jnp.where(c == 0, CH0, CH1)
        cht_half = jnp.where(c == 0, CH0 // 2, CH1 // 2)

        def base_of(ch):
            return ebase + lax.rem(ch, cht) * K

        def start_idx(b, ch):
            bs = base_of(ch)
            pltpu.async_copy(rowi.at[pl.ds(bs, K)], rowv.at[b], semi[b])
            pltpu.async_copy(coli.at[pl.ds(bs, K)], colv.at[b], semi[b])

        def wait_idx(b):
            pltpu.make_async_copy(rowi.at[pl.ds(0, K)], rowv.at[b],
                                  semi[b]).wait()
            pltpu.make_async_copy(coli.at[pl.ds(0, K)], colv.at[b],
                                  semi[b]).wait()

        def start_gather(sl, b):
            pltpu.async_copy(tabs[sl].at[rowv.at[b]], datav.at[b], semg[b])

        def wait_gather(sl, b):
            pltpu.make_async_copy(tabs[sl].at[rowv.at[b]], datav.at[b],
                                  semg[b]).wait()

        for sl in range(S):
            pltpu.sync_copy(zeros_hbm, datav.at[0])
            for z in range(STRIPE // K):
                pltpu.sync_copy(datav.at[0],
                                acc.at[pl.ds(s * STRIPE + z * K, K), :])
            start_idx(0, 0)
            start_idx(1, 1)
            wait_idx(0)
            start_gather(sl, 0)
            plsc.subcore_barrier()

            def pair(gi, carry):
                for b in (0, 1):
                    ch = 2 * gi + b
                    wait_idx(b ^ 1)
                    start_gather(sl, b ^ 1)
                    wait_gather(sl, b)
                    pltpu.sync_copy(datav.at[b], acc.at[colv.at[b]],
                                    add=True)
                    start_idx(b, ch + 2)
                return carry

            lax.fori_loop(0, cht_half, pair, 0)
            wait_gather(sl, 0)
            wait_idx(1)
            plsc.subcore_barrier()
            pltpu.sync_copy(acc.at[pl.ds(s * STRIPE, STRIPE), :],
                            outs[sl].at[c, pl.ds(s * STRIPE, STRIPE), :])

    return body


def _sc_degree():
    mesh = plsc.VectorSubcoreMesh(core_axis_name="c", subcore_axis_name="s")
    out_type = jax.ShapeDtypeStruct((NC, NPAD, 128), _F32)
    scratch = [
        pltpu.VMEM_SHARED((NPAD, 128), _F32),
        pltpu.VMEM((K,), jnp.int32),
        pltpu.VMEM((K, 128), _F32),
    ]

    @functools.partial(pl.kernel, out_type=out_type, mesh=mesh,
                       scratch_types=scratch)
    def body(coli, zeros_hbm, ones_hbm, out, acc, colv, datav):
        c = lax.axis_index("c")
        s = lax.axis_index("s")
        wid = c * NS + s
        ebase = wid * (K * CH)
        pltpu.sync_copy(zeros_hbm, datav)
        for z in range(STRIPE // K):
            pltpu.sync_copy(datav, acc.at[pl.ds(s * STRIPE + z * K, K), :])
        plsc.subcore_barrier()
        pltpu.sync_copy(ones_hbm, datav)

        def chunk(ch, carry):
            base = ebase + ch * K
            pltpu.sync_copy(coli.at[pl.ds(base, K)], colv)
            pltpu.sync_copy(datav, acc.at[colv], add=True)
            return carry

        lax.fori_loop(0, CH, chunk, 0)
        plsc.subcore_barrier()
        pltpu.sync_copy(acc.at[pl.ds(s * STRIPE, STRIPE), :],
                        out.at[c, pl.ds(s * STRIPE, STRIPE), :])

    return body


_SEQ = pltpu.CompilerParams(dimension_semantics=("arbitrary",))


def _row_block(j=None):
    if j is None:
        return pl.BlockSpec((BN_, None), lambda i: (i, 0))
    return None


def _stats(a, i, sums_ref, bn_block):
    rows = i * BN_ + lax.broadcasted_iota(jnp.int32, (bn_block, 1), 0)
    mask = rows < N
    am = jnp.where(mask, a, 0.0)
    am2 = jnp.where(mask, a * a, 0.0)
    part = jnp.concatenate([jnp.sum(am, axis=0, keepdims=True),
                            jnp.sum(am2, axis=0, keepdims=True)], axis=0)

    @pl.when(i == 0)
    def _():
        sums_ref[...] = part

    @pl.when(i > 0)
    def _():
        sums_ref[...] = sums_ref[...] + part


def _k_pre(xp, degc):
    def body(x_ref, deg_ref, dis_ref, xt0_ref, xt1_ref):
        d = lax.rsqrt(deg_ref[...] + 1.0)
        dis_ref[...] = d
        xt = x_ref[...] * d
        xt0_ref[...] = xt[:, :128]
        xt1_ref[...] = xt[:, 128:]

    return pl.pallas_call(
        body,
        grid=(NBLK,),
        in_specs=[pl.BlockSpec((BN_, DIN), lambda i: (i, 0)),
                  pl.BlockSpec((BN_, 1), lambda i: (i, 0))],
        out_specs=[pl.BlockSpec((BN_, 1), lambda i: (i, 0)),
                   pl.BlockSpec((BN_, 128), lambda i: (i, 0)),
                   pl.BlockSpec((BN_, 128), lambda i: (i, 0))],
        out_shape=[jax.ShapeDtypeStruct((NPAD, 1), _F32),
                   jax.ShapeDtypeStruct((NPAD, 128), _F32),
                   jax.ShapeDtypeStruct((NPAD, 128), _F32)],
        compiler_params=_SEQ,
    )(xp, degc)


def _k_layer1(u0, u1, xt0, xt1, dis, W1, b1):
    def body(u0_ref, u1_ref, xt0_ref, xt1_ref, dis_ref, w_ref, b_ref,
             a_ref, sums_ref):
        i = pl.program_id(0)
        d = dis_ref[...]
        m0 = d * (u0_ref[0] + u0_ref[1] + xt0_ref[...])
        m1 = d * (u1_ref[0] + u1_ref[1] + xt1_ref[...])
        w = w_ref[...]
        a = (lax.dot_general(m0, w[:, :128], (((1,), (1,)), ((), ())),
                             preferred_element_type=_F32)
             + lax.dot_general(m1, w[:, 128:], (((1,), (1,)), ((), ())),
                               preferred_element_type=_F32)
             + b_ref[...])
        a_ref[...] = a
        _stats(a, i, sums_ref, BN_)

    return pl.pallas_call(
        body,
        grid=(NBLK,),
        in_specs=[pl.BlockSpec((NC, BN_, 128), lambda i: (0, i, 0)),
                  pl.BlockSpec((NC, BN_, 128), lambda i: (0, i, 0)),
                  pl.BlockSpec((BN_, 128), lambda i: (i, 0)),
                  pl.BlockSpec((BN_, 128), lambda i: (i, 0)),
                  pl.BlockSpec((BN_, 1), lambda i: (i, 0)),
                  pl.BlockSpec((H, DIN), lambda i: (0, 0)),
                  pl.BlockSpec((1, H), lambda i: (0, 0))],
        out_specs=[pl.BlockSpec((BN_, H), lambda i: (i, 0)),
                   pl.BlockSpec((2, H), lambda i: (0, 0))],
        out_shape=[jax.ShapeDtypeStruct((NPAD, H), _F32),
                   jax.ShapeDtypeStruct((2, H), _F32)],
        compiler_params=_SEQ,
    )(u0, u1, xt0, xt1, dis, W1, b1)


def _k_act(a, scale, shift, dis, Wn, resid=None, emit_y=True):
    hin = a.shape[1]
    sout = Wn.shape[0] // 128
    has_res = resid is not None

    def body(*refs):
        a_ref, sc_ref, sh_ref, dis_ref = refs[:4]
        idx = 4
        if has_res:
            res_ref = refs[idx]; idx += 1
        w_ref = refs[idx]; idx += 1
        outs = refs[idx:]
        y = jnp.maximum(a_ref[...] * sc_ref[...] + sh_ref[...], 0.0)
        if has_res:
            y = y + res_ref[...]
        oidx = 0
        if emit_y:
            outs[0][...] = y
            oidx = 1
        m = dis_ref[...] * y
        w = w_ref[...]
        for j in range(sout):
            outs[oidx + j][...] = lax.dot_general(
                m, w[j * 128:(j + 1) * 128, :], (((1,), (1,)), ((), ())),
                preferred_element_type=_F32)

    in_specs = [pl.BlockSpec((BN_, hin), lambda i: (i, 0)),
                pl.BlockSpec((1, hin), lambda i: (0, 0)),
                pl.BlockSpec((1, hin), lambda i: (0, 0)),
                pl.BlockSpec((BN_, 1), lambda i: (i, 0))]
    args = [a, scale, shift, dis]
    if has_res:
        in_specs.append(pl.BlockSpec((BN_, hin), lambda i: (i, 0)))
        args.append(resid)
    in_specs.append(pl.BlockSpec(Wn.shape, lambda i: (0, 0)))
    args.append(Wn)
    out_specs, out_shape = [], []
    if emit_y:
        out_specs.append(pl.BlockSpec((BN_, hin), lambda i: (i, 0)))
        out_shape.append(jax.ShapeDtypeStruct((NPAD, hin), _F32))
    for _ in range(sout):
        out_specs.append(pl.BlockSpec((BN_, 128), lambda i: (i, 0)))
        out_shape.append(jax.ShapeDtypeStruct((NPAD, 128), _F32))

    return pl.pallas_call(
        body, grid=(NBLK,), in_specs=in_specs, out_specs=out_specs,
        out_shape=out_shape, compiler_params=_SEQ,
    )(*args)


def _k_post(svals, pvals, dis, b):
    S = len(pvals)
    hw = S * 128

    def body(*refs):
        s_refs = refs[:S]
        p_refs = refs[S:2 * S]
        dis_ref, b_ref = refs[2 * S], refs[2 * S + 1]
        a_ref, sums_ref = refs[2 * S + 2], refs[2 * S + 3]
        i = pl.program_id(0)
        d = dis_ref[...]
        parts = [d * (s_refs[j][0] + s_refs[j][1] + p_refs[j][...])
                 for j in range(S)]
        a = jnp.concatenate(parts, axis=1) + b_ref[...]
        a_ref[...] = a
        _stats(a, i, sums_ref, BN_)

    in_specs = ([pl.BlockSpec((NC, BN_, 128), lambda i: (0, i, 0))] * S
                + [pl.BlockSpec((BN_, 128), lambda i: (i, 0))] * S
                + [pl.BlockSpec((BN_, 1), lambda i: (i, 0)),
                   pl.BlockSpec((1, hw), lambda i: (0, 0))])

    return pl.pallas_call(
        body, grid=(NBLK,), in_specs=in_specs,
        out_specs=[pl.BlockSpec((BN_, hw), lambda i: (i, 0)),
                   pl.BlockSpec((2, hw), lambda i: (0, 0))],
        out_shape=[jax.ShapeDtypeStruct((NPAD, hw), _F32),
                   jax.ShapeDtypeStruct((2, hw), _F32)],
        compiler_params=_SEQ,
    )(*(list(svals) + list(pvals) + [dis, b]))


def _k_pool(a4, scale, shift, batchp):
    def body(a_ref, sc_ref, sh_ref, b_ref, psum_ref, pcnt_ref):
        i = pl.program_id(0)
        y = jnp.maximum(a_ref[...] * sc_ref[...] + sh_ref[...], 0.0)
        bv = b_ref[...]
        oh = (bv == lax.broadcasted_iota(jnp.int32, (BN_, G), 1)).astype(_F32)
        ps = lax.dot_general(oh, y, (((0,), (0,)), ((), ())),
                             preferred_element_type=_F32)
        pc = jnp.sum(oh, axis=0)[:, None]

        @pl.when(i == 0)
        def _():
            psum_ref[...] = ps
            pcnt_ref[...] = pc

        @pl.when(i > 0)
        def _():
            psum_ref[...] = psum_ref[...] + ps
            pcnt_ref[...] = pcnt_ref[...] + pc

    hw = a4.shape[1]
    return pl.pallas_call(
        body,
        grid=(NBLK,),
        in_specs=[pl.BlockSpec((BN_, hw), lambda i: (i, 0)),
                  pl.BlockSpec((1, hw), lambda i: (0, 0)),
                  pl.BlockSpec((1, hw), lambda i: (0, 0)),
                  pl.BlockSpec((BN_, 1), lambda i: (i, 0))],
        out_specs=[pl.BlockSpec((G, hw), lambda i: (0, 0)),
                   pl.BlockSpec((G, 1), lambda i: (0, 0))],
        out_shape=[jax.ShapeDtypeStruct((G, hw), _F32),
                   jax.ShapeDtypeStruct((G, 1), _F32)],
        compiler_params=_SEQ,
    )(a4, scale, shift, batchp)


def _k_classifier(psum, pcnt, Wc1, bc1, gc1, bec1, Wc2, bc2, gc2, bec2,
                  Wc3, bc3):
    def body(psum_ref, pcnt_ref, w1_ref, b1_ref, g1_ref, be1_ref,
             w2_ref, b2_ref, g2_ref, be2_ref, w3_ref, b3_ref, out_ref):
        pooled = psum_ref[...] / jnp.maximum(pcnt_ref[...], 1.0)

        def dense(hh, w_ref, b_ref):
            return lax.dot_general(hh, w_ref[...], (((1,), (1,)), ((), ())),
                                   preferred_element_type=_F32) + b_ref[...]

        def bn(hh, g_ref, be_ref):
            m = jnp.mean(hh, axis=0, keepdims=True)
            v = jnp.mean((hh - m) ** 2, axis=0, keepdims=True)
            return g_ref[...] * (hh - m) * lax.rsqrt(v + 1e-5) + be_ref[...]

        h = jnp.maximum(dense(pooled, w1_ref, b1_ref), 0.0)
        h = bn(h, g1_ref, be1_ref)
        h = jnp.maximum(dense(h, w2_ref, b2_ref), 0.0)
        h = bn(h, g2_ref, be2_ref)
        logits = dense(h, w3_ref, b3_ref)
        mx = jnp.max(logits, axis=1, keepdims=True)
        lse = jnp.log(jnp.sum(jnp.exp(logits - mx), axis=1,
                              keepdims=True)) + mx
        out_ref[...] = logits - lse

    return pl.pallas_call(
        body,
        out_shape=jax.ShapeDtypeStruct((G, NCLS), _F32),
    )(psum, pcnt, Wc1, bc1, gc1, bec1, Wc2, bc2, gc2, bec2, Wc3, bc3)


def _fold_bn(sums, g, be):
    mean = sums[0] / N
    var = sums[1] / N - mean * mean
    scale = g * lax.rsqrt(var + 1e-5)
    shift = be - mean * scale
    return scale[None], shift[None]


def kernel(x, edge_index, batch, W1, b1, W2, b2, W3, b3, W4, b4,
           g1, be1, g2, be2, g3, be3, g4, be4,
           Wc1, bc1, gc1, bec1, Wc2, bc2, gc2, bec2, Wc3, bc3):
    xp = jnp.pad(x, ((0, NPAD - N), (0, 0)))
    batchp = jnp.pad(batch, (0, NPAD - N), constant_values=G)[:, None]
    rowp = jnp.pad(edge_index[0], (0, EPAD - E))
    colp = jnp.pad(edge_index[1], (0, EPAD - E), constant_values=GARBAGE)
    zeros128 = jnp.zeros((K, 128), _F32)
    ones128 = jnp.ones((K, 128), _F32)

    scat2 = _sc_scatter_rows(2, CH_A, CH_B)
    scat4 = _sc_scatter_rows(4, CH_A, CH_B)

    dego = _sc_degree()(colp, zeros128, ones128)
    degc = dego[0, :, :1] + dego[1, :, :1]

    dis, xt0, xt1 = _k_pre(xp, degc)
    u0, u1 = scat2(rowp, colp, zeros128, xt0, xt1)
    a1, sums1 = _k_layer1(u0, u1, xt0, xt1, dis, W1, b1[None])
    sc1, sh1 = _fold_bn(sums1, g1, be1)

    y1, p0, p1_, p2_, p3_ = _k_act(a1, sc1, sh1, dis, W2, emit_y=True)
    s = scat4(rowp, colp, zeros128, p0, p1_, p2_, p3_)
    a2, sums2 = _k_post(s, (p0, p1_, p2_, p3_), dis, b2[None])
    sc2, sh2 = _fold_bn(sums2, g2, be2)

    y2, q0, q1, q2, q3 = _k_act(a2, sc2, sh2, dis, W3, resid=y1, emit_y=True)
    s = scat4(rowp, colp, zeros128, q0, q1, q2, q3)
    a3, sums3 = _k_post(s, (q0, q1, q2, q3), dis, b3[None])
    sc3, sh3 = _fold_bn(sums3, g3, be3)

    r0, r1 = _k_act(a3, sc3, sh3, dis, W4, resid=y2, emit_y=False)
    s = scat2(rowp, colp, zeros128, r0, r1)
    a4, sums4 = _k_post(s, (r0, r1), dis, b4[None])
    sc4, sh4 = _fold_bn(sums4, g4, be4)

    psum, pcnt = _k_pool(a4, sc4, sh4, batchp)
    return _k_classifier(psum, pcnt, Wc1, bc1[None], gc1[None], bec1[None],
                         Wc2, bc2[None], gc2[None], bec2[None],
                         Wc3, bc3[None])


# speedup vs baseline: 1.5872x; 1.5872x over previous
import functools

import jax
import jax.numpy as jnp
from jax import lax
from jax.experimental import pallas as pl
from jax.experimental.pallas import tpu as pltpu
from jax.experimental.pallas import tpu_sc as plsc

N = 10000
E = 160000
DIN = 256
H = 512
G = 64
NCLS = 5

NPAD = 10240
BN_ = 512
NBLK = NPAD // BN_
NC, NS = 2, 16
K = 128
CH = 40
CH_A, CH_B = 78, 2
EPAD = NC * NS * K * CH
STRIPE = NPAD // NS
GARBAGE = 10016

_F32 = jnp.float32


def _sc_scatter_rows(S):
    mesh = plsc.VectorSubcoreMesh(core_axis_name="c", subcore_axis_name="s")
    out_type = [jax.ShapeDtypeStruct((NPAD, 128), _F32) for _ in range(S)]
    scratch = [
        pltpu.VMEM_SHARED((NPAD, 128), _F32),
        pltpu.VMEM((2, K), jnp.int32),
        pltpu.VMEM((2, K), jnp.int32),
        pltpu.VMEM((2, K, 128), _F32),
        pltpu.SemaphoreType.DMA,
        pltpu.SemaphoreType.DMA,
        pltpu.SemaphoreType.DMA,
        pltpu.SemaphoreType.DMA,
    ]
    CHT = EPAD // (NS * K)

    @functools.partial(pl.kernel, out_type=out_type, mesh=mesh,
                       scratch_types=scratch)
    def body(rowi, coli, zeros_hbm, *rest):
        tabs = rest[:S]
        outs = rest[S:2 * S]
        acc, rowv, colv, datav, si0, si1, sg0, sg1 = rest[2 * S:]
        semi = (si0, si1)
        semg = (sg0, sg1)
        c = lax.axis_index("c")
        s = lax.axis_index("s")
        ebase = s * (K * CHT)

        def start_idx(b, ch):
            bs = ebase + (ch % CHT) * K
            pltpu.async_copy(rowi.at[pl.ds(bs, K)], rowv.at[b], semi[b])
            pltpu.async_copy(coli.at[pl.ds(bs, K)], colv.at[b], semi[b])

        def wait_idx(b):
            pltpu.make_async_copy(rowi.at[pl.ds(0, K)], rowv.at[b],
                                  semi[b]).wait()
            pltpu.make_async_copy(coli.at[pl.ds(0, K)], colv.at[b],
                                  semi[b]).wait()

        def start_gather(sl, b):
            pltpu.async_copy(tabs[sl].at[rowv.at[b]], datav.at[b], semg[b])

        def wait_gather(sl, b):
            pltpu.make_async_copy(tabs[sl].at[rowv.at[b]], datav.at[b],
                                  semg[b]).wait()

        def run_slice(sl):
            pltpu.sync_copy(zeros_hbm, datav.at[0])
            for z in range(STRIPE // K):
                pltpu.sync_copy(datav.at[0],
                                acc.at[pl.ds(s * STRIPE + z * K, K), :])
            start_idx(0, 0)
            start_idx(1, 1)
            wait_idx(0)
            start_gather(sl, 0)
            plsc.subcore_barrier()

            def pair(gi, carry):
                for b in (0, 1):
                    wait_idx(b ^ 1)
                    start_gather(sl, b ^ 1)
                    wait_gather(sl, b)
                    pltpu.sync_copy(datav.at[b], acc.at[colv.at[b]],
                                    add=True)
                    start_idx(b, 2 * gi + b + 2)
                return carry

            lax.fori_loop(0, CHT // 2, pair, 0)
            wait_gather(sl, 0)
            wait_idx(1)
            plsc.subcore_barrier()
            pltpu.sync_copy(acc.at[pl.ds(s * STRIPE, STRIPE), :],
                            outs[sl].at[pl.ds(s * STRIPE, STRIPE), :])

        for half in range(NC):
            @pl.when(c == half)
            def _():
                for sl in range(half, S, NC):
                    run_slice(sl)

    return body


def _sc_degree():
    mesh = plsc.VectorSubcoreMesh(core_axis_name="c", subcore_axis_name="s")
    out_type = jax.ShapeDtypeStruct((NC, NPAD, 128), _F32)
    scratch = [
        pltpu.VMEM_SHARED((NPAD, 128), _F32),
        pltpu.VMEM((K,), jnp.int32),
        pltpu.VMEM((K, 128), _F32),
    ]

    @functools.partial(pl.kernel, out_type=out_type, mesh=mesh,
                       scratch_types=scratch)
    def body(coli, zeros_hbm, ones_hbm, out, acc, colv, datav):
        c = lax.axis_index("c")
        s = lax.axis_index("s")
        wid = c * NS + s
        ebase = wid * (K * CH)
        pltpu.sync_copy(zeros_hbm, datav)
        for z in range(STRIPE // K):
            pltpu.sync_copy(datav, acc.at[pl.ds(s * STRIPE + z * K, K), :])
        plsc.subcore_barrier()
        pltpu.sync_copy(ones_hbm, datav)

        def chunk(ch, carry):
            base = ebase + ch * K
            pltpu.sync_copy(coli.at[pl.ds(base, K)], colv)
            pltpu.sync_copy(datav, acc.at[colv], add=True)
            return carry

        lax.fori_loop(0, CH, chunk, 0)
        plsc.subcore_barrier()
        pltpu.sync_copy(acc.at[pl.ds(s * STRIPE, STRIPE), :],
                        out.at[c, pl.ds(s * STRIPE, STRIPE), :])

    return body


_SEQ = pltpu.CompilerParams(dimension_semantics=("arbitrary",))


def _row_block(j=None):
    if j is None:
        return pl.BlockSpec((BN_, None), lambda i: (i, 0))
    return None


def _stats(a, i, sums_ref, bn_block):
    rows = i * BN_ + lax.broadcasted_iota(jnp.int32, (bn_block, 1), 0)
    mask = rows < N
    am = jnp.where(mask, a, 0.0)
    am2 = jnp.where(mask, a * a, 0.0)
    part = jnp.concatenate([jnp.sum(am, axis=0, keepdims=True),
                            jnp.sum(am2, axis=0, keepdims=True)], axis=0)

    @pl.when(i == 0)
    def _():
        sums_ref[...] = part

    @pl.when(i > 0)
    def _():
        sums_ref[...] = sums_ref[...] + part


def _k_pre(xp, degc):
    def body(x_ref, deg_ref, dis_ref, xt0_ref, xt1_ref):
        d = lax.rsqrt(deg_ref[...] + 1.0)
        dis_ref[...] = d
        xt = x_ref[...] * d
        xt0_ref[...] = xt[:, :128]
        xt1_ref[...] = xt[:, 128:]

    return pl.pallas_call(
        body,
        grid=(NBLK,),
        in_specs=[pl.BlockSpec((BN_, DIN), lambda i: (i, 0)),
                  pl.BlockSpec((BN_, 1), lambda i: (i, 0))],
        out_specs=[pl.BlockSpec((BN_, 1), lambda i: (i, 0)),
                   pl.BlockSpec((BN_, 128), lambda i: (i, 0)),
                   pl.BlockSpec((BN_, 128), lambda i: (i, 0))],
        out_shape=[jax.ShapeDtypeStruct((NPAD, 1), _F32),
                   jax.ShapeDtypeStruct((NPAD, 128), _F32),
                   jax.ShapeDtypeStruct((NPAD, 128), _F32)],
        compiler_params=_SEQ,
    )(xp, degc)


def _k_layer1(u0, u1, xt0, xt1, dis, W1, b1):
    def body(u0_ref, u1_ref, xt0_ref, xt1_ref, dis_ref, w_ref, b_ref,
             a_ref, sums_ref):
        i = pl.program_id(0)
        d = dis_ref[...]
        m0 = d * (u0_ref[...] + xt0_ref[...])
        m1 = d * (u1_ref[...] + xt1_ref[...])
        w = w_ref[...]
        a = (lax.dot_general(m0, w[:, :128], (((1,), (1,)), ((), ())),
                             preferred_element_type=_F32)
             + lax.dot_general(m1, w[:, 128:], (((1,), (1,)), ((), ())),
                               preferred_element_type=_F32)
             + b_ref[...])
        a_ref[...] = a
        _stats(a, i, sums_ref, BN_)

    return pl.pallas_call(
        body,
        grid=(NBLK,),
        in_specs=[pl.BlockSpec((BN_, 128), lambda i: (i, 0)),
                  pl.BlockSpec((BN_, 128), lambda i: (i, 0)),
                  pl.BlockSpec((BN_, 128), lambda i: (i, 0)),
                  pl.BlockSpec((BN_, 128), lambda i: (i, 0)),
                  pl.BlockSpec((BN_, 1), lambda i: (i, 0)),
                  pl.BlockSpec((H, DIN), lambda i: (0, 0)),
                  pl.BlockSpec((1, H), lambda i: (0, 0))],
        out_specs=[pl.BlockSpec((BN_, H), lambda i: (i, 0)),
                   pl.BlockSpec((2, H), lambda i: (0, 0))],
        out_shape=[jax.ShapeDtypeStruct((NPAD, H), _F32),
                   jax.ShapeDtypeStruct((2, H), _F32)],
        compiler_params=_SEQ,
    )(u0, u1, xt0, xt1, dis, W1, b1)


def _k_act(a, scale, shift, dis, Wn, resid=None, emit_y=True):
    hin = a.shape[1]
    sout = Wn.shape[0] // 128
    has_res = resid is not None

    def body(*refs):
        a_ref, sc_ref, sh_ref, dis_ref = refs[:4]
        idx = 4
        if has_res:
            res_ref = refs[idx]; idx += 1
        w_ref = refs[idx]; idx += 1
        outs = refs[idx:]
        y = jnp.maximum(a_ref[...] * sc_ref[...] + sh_ref[...], 0.0)
        if has_res:
            y = y + res_ref[...]
        oidx = 0
        if emit_y:
            outs[0][...] = y
            oidx = 1
        m = dis_ref[...] * y
        w = w_ref[...]
        for j in range(sout):
            outs[oidx + j][...] = lax.dot_general(
                m, w[j * 128:(j + 1) * 128, :], (((1,), (1,)), ((), ())),
                preferred_element_type=_F32)

    in_specs = [pl.BlockSpec((BN_, hin), lambda i: (i, 0)),
                pl.BlockSpec((1, hin), lambda i: (0, 0)),
                pl.BlockSpec((1, hin), lambda i: (0, 0)),
                pl.BlockSpec((BN_, 1), lambda i: (i, 0))]
    args = [a, scale, shift, dis]
    if has_res:
        in_specs.append(pl.BlockSpec((BN_, hin), lambda i: (i, 0)))
        args.append(resid)
    in_specs.append(pl.BlockSpec(Wn.shape, lambda i: (0, 0)))
    args.append(Wn)
    out_specs, out_shape = [], []
    if emit_y:
        out_specs.append(pl.BlockSpec((BN_, hin), lambda i: (i, 0)))
        out_shape.append(jax.ShapeDtypeStruct((NPAD, hin), _F32))
    for _ in range(sout):
        out_specs.append(pl.BlockSpec((BN_, 128), lambda i: (i, 0)))
        out_shape.append(jax.ShapeDtypeStruct((NPAD, 128), _F32))

    return pl.pallas_call(
        body, grid=(NBLK,), in_specs=in_specs, out_specs=out_specs,
        out_shape=out_shape, compiler_params=_SEQ,
    )(*args)


def _k_post(svals, pvals, dis, b):
    S = len(pvals)
    hw = S * 128

    def body(*refs):
        s_refs = refs[:S]
        p_refs = refs[S:2 * S]
        dis_ref, b_ref = refs[2 * S], refs[2 * S + 1]
        a_ref, sums_ref = refs[2 * S + 2], refs[2 * S + 3]
        i = pl.program_id(0)
        d = dis_ref[...]
        parts = [d * (s_refs[j][...] + p_refs[j][...])
                 for j in range(S)]
        a = jnp.concatenate(parts, axis=1) + b_ref[...]
        a_ref[...] = a
        _stats(a, i, sums_ref, BN_)

    in_specs = ([pl.BlockSpec((BN_, 128), lambda i: (i, 0))] * S
                + [pl.BlockSpec((BN_, 128), lambda i: (i, 0))] * S
                + [pl.BlockSpec((BN_, 1), lambda i: (i, 0)),
                   pl.BlockSpec((1, hw), lambda i: (0, 0))])

    return pl.pallas_call(
        body, grid=(NBLK,), in_specs=in_specs,
        out_specs=[pl.BlockSpec((BN_, hw), lambda i: (i, 0)),
                   pl.BlockSpec((2, hw), lambda i: (0, 0))],
        out_shape=[jax.ShapeDtypeStruct((NPAD, hw), _F32),
                   jax.ShapeDtypeStruct((2, hw), _F32)],
        compiler_params=_SEQ,
    )(*(list(svals) + list(pvals) + [dis, b]))


def _k_pool(a4, scale, shift, batchp):
    def body(a_ref, sc_ref, sh_ref, b_ref, psum_ref, pcnt_ref):
        i = pl.program_id(0)
        y = jnp.maximum(a_ref[...] * sc_ref[...] + sh_ref[...], 0.0)
        bv = b_ref[...]
        oh = (bv == lax.broadcasted_iota(jnp.int32, (BN_, G), 1)).astype(_F32)
        ps = lax.dot_general(oh, y, (((0,), (0,)), ((), ())),
                             preferred_element_type=_F32)
        pc = jnp.sum(oh, axis=0)[:, None]

        @pl.when(i == 0)
        def _():
            psum_ref[...] = ps
            pcnt_ref[...] = pc

        @pl.when(i > 0)
        def _():
            psum_ref[...] = psum_ref[...] + ps
            pcnt_ref[...] = pcnt_ref[...] + pc

    hw = a4.shape[1]
    return pl.pallas_call(
        body,
        grid=(NBLK,),
        in_specs=[pl.BlockSpec((BN_, hw), lambda i: (i, 0)),
                  pl.BlockSpec((1, hw), lambda i: (0, 0)),
                  pl.BlockSpec((1, hw), lambda i: (0, 0)),
                  pl.BlockSpec((BN_, 1), lambda i: (i, 0))],
        out_specs=[pl.BlockSpec((G, hw), lambda i: (0, 0)),
                   pl.BlockSpec((G, 1), lambda i: (0, 0))],
        out_shape=[jax.ShapeDtypeStruct((G, hw), _F32),
                   jax.ShapeDtypeStruct((G, 1), _F32)],
        compiler_params=_SEQ,
    )(a4, scale, shift, batchp)


def _k_classifier(psum, pcnt, Wc1, bc1, gc1, bec1, Wc2, bc2, gc2, bec2,
                  Wc3, bc3):
    def body(psum_ref, pcnt_ref, w1_ref, b1_ref, g1_ref, be1_ref,
             w2_ref, b2_ref, g2_ref, be2_ref, w3_ref, b3_ref, out_ref):
        pooled = psum_ref[...] / jnp.maximum(pcnt_ref[...], 1.0)

        def dense(hh, w_ref, b_ref):
            return lax.dot_general(hh, w_ref[...], (((1,), (1,)), ((), ())),
                                   preferred_element_type=_F32) + b_ref[...]

        def bn(hh, g_ref, be_ref):
            m = jnp.mean(hh, axis=0, keepdims=True)
            v = jnp.mean((hh - m) ** 2, axis=0, keepdims=True)
            return g_ref[...] * (hh - m) * lax.rsqrt(v + 1e-5) + be_ref[...]

        h = jnp.maximum(dense(pooled, w1_ref, b1_ref), 0.0)
        h = bn(h, g1_ref, be1_ref)
        h = jnp.maximum(dense(h, w2_ref, b2_ref), 0.0)
        h = bn(h, g2_ref, be2_ref)
        logits = dense(h, w3_ref, b3_ref)
        mx = jnp.max(logits, axis=1, keepdims=True)
        lse = jnp.log(jnp.sum(jnp.exp(logits - mx), axis=1,
                              keepdims=True)) + mx
        out_ref[...] = logits - lse

    return pl.pallas_call(
        body,
        out_shape=jax.ShapeDtypeStruct((G, NCLS), _F32),
    )(psum, pcnt, Wc1, bc1, gc1, bec1, Wc2, bc2, gc2, bec2, Wc3, bc3)


def _fold_bn(sums, g, be):
    mean = sums[0] / N
    var = sums[1] / N - mean * mean
    scale = g * lax.rsqrt(var + 1e-5)
    shift = be - mean * scale
    return scale[None], shift[None]


def kernel(x, edge_index, batch, W1, b1, W2, b2, W3, b3, W4, b4,
           g1, be1, g2, be2, g3, be3, g4, be4,
           Wc1, bc1, gc1, bec1, Wc2, bc2, gc2, bec2, Wc3, bc3):
    xp = jnp.pad(x, ((0, NPAD - N), (0, 0)))
    batchp = jnp.pad(batch, (0, NPAD - N), constant_values=G)[:, None]
    rowp = jnp.pad(edge_index[0], (0, EPAD - E))
    colp = jnp.pad(edge_index[1], (0, EPAD - E), constant_values=GARBAGE)
    zeros128 = jnp.zeros((K, 128), _F32)
    ones128 = jnp.ones((K, 128), _F32)

    scat2 = _sc_scatter_rows(2)
    scat4 = _sc_scatter_rows(4)

    dego = _sc_degree()(colp, zeros128, ones128)
    degc = dego[0, :, :1] + dego[1, :, :1]

    dis, xt0, xt1 = _k_pre(xp, degc)
    u0, u1 = scat2(rowp, colp, zeros128, xt0, xt1)
    a1, sums1 = _k_layer1(u0, u1, xt0, xt1, dis, W1, b1[None])
    sc1, sh1 = _fold_bn(sums1, g1, be1)

    y1, p0, p1_, p2_, p3_ = _k_act(a1, sc1, sh1, dis, W2, emit_y=True)
    s = scat4(rowp, colp, zeros128, p0, p1_, p2_, p3_)
    a2, sums2 = _k_post(s, (p0, p1_, p2_, p3_), dis, b2[None])
    sc2, sh2 = _fold_bn(sums2, g2, be2)

    y2, q0, q1, q2, q3 = _k_act(a2, sc2, sh2, dis, W3, resid=y1, emit_y=True)
    s = scat4(rowp, colp, zeros128, q0, q1, q2, q3)
    a3, sums3 = _k_post(s, (q0, q1, q2, q3), dis, b3[None])
    sc3, sh3 = _fold_bn(sums3, g3, be3)

    r0, r1 = _k_act(a3, sc3, sh3, dis, W4, resid=y2, emit_y=False)
    s = scat2(rowp, colp, zeros128, r0, r1)
    a4, sums4 = _k_post(s, (r0, r1), dis, b4[None])
    sc4, sh4 = _fold_bn(sums4, g4, be4)

    psum, pcnt = _k_pool(a4, sc4, sh4, batchp)
    return _k_classifier(psum, pcnt, Wc1, bc1[None], gc1[None], bec1[None],
                         Wc2, bc2[None], gc2[None], bec2[None],
                         Wc3, bc3[None])
